# Initial kernel scaffold; baseline (speedup 1.0000x reference)
import jax
import jax.numpy as jnp
from jax.experimental import pallas as pl


def kernel(x, edge_index, W1, b1, W2, b2, Wv, bv, Wt, bt):
    raise NotImplementedError("write your pallas kernel here")


# speedup vs baseline: 16.5510x; 16.5510x over previous
import functools

import jax
import jax.numpy as jnp
from jax import lax
from jax.experimental import pallas as pl
from jax.experimental.pallas import tpu as pltpu
from jax.experimental.pallas import tpu_sc as plsc

N = 10000
D = 128
E = 320000

NC = 2
NS = 16
C = 128
NCHUNK = E // C
HALF = NCHUNK // NC
NPAD = 10240
ROWS_PER_TILE = NPAD // NS

_MESH = plsc.VectorSubcoreMesh(core_axis_name="c", subcore_axis_name="s")


def _cnt_body(dst_hbm, out_hbm, cnt_sp, zbuf, ones_v, didx):
    c = lax.axis_index("c")
    s = lax.axis_index("s")

    zeros16 = jnp.zeros((16,), jnp.float32)
    ones16 = jnp.ones((16,), jnp.float32)

    def _zb(i, _):
        zbuf[pl.ds(i * 16, 16)] = zeros16
        return 0

    lax.fori_loop(0, 2000 // 16, _zb, 0)

    def _ob(i, _):
        ones_v[pl.ds(i * 16, 16)] = ones16
        return 0

    lax.fori_loop(0, C // 16, _ob, 0)

    @pl.when(s < 5)
    def _():
        pltpu.sync_copy(zbuf, cnt_sp.at[pl.ds(s * 2000, 2000)])

    plsc.subcore_barrier()

    def _body(k, _):
        i = s + k * NS
        off = (c * HALF + i) * C
        pltpu.sync_copy(dst_hbm.at[pl.ds(off, C)], didx)
        pltpu.sync_copy(ones_v, cnt_sp.at[didx], add=True)
        return 0

    nmine = (HALF - s + NS - 1) // NS
    lax.fori_loop(0, nmine, _body, 0)

    plsc.subcore_barrier()

    @pl.when(s < 10)
    def _():
        pltpu.sync_copy(cnt_sp.at[pl.ds(s * 1000, 1000)],
                        zbuf.at[pl.ds(0, 1000)])
        pltpu.sync_copy(zbuf.at[pl.ds(0, 1000)],
                        out_hbm.at[pl.ds(c * N + s * 1000, 1000)])


@functools.partial(
    pl.kernel,
    out_type=jax.ShapeDtypeStruct((NC * N,), jnp.float32),
    mesh=_MESH,
    scratch_types=[
        pltpu.VMEM_SHARED((N,), jnp.float32),
        pltpu.VMEM((2000,), jnp.float32),
        pltpu.VMEM((C,), jnp.float32),
        pltpu.VMEM((C,), jnp.int32),
    ],
    name="sc_degree_count",
)
def _sc_cnt(dst_hbm, out_hbm, cnt_sp, zbuf, ones_v, didx):
    _cnt_body(dst_hbm, out_hbm, cnt_sp, zbuf, ones_v, didx)


def _edge_body(hs_hbm, src_hbm, dst_hbm, out_hbm, acc_sp, rows_v, sidx, didx,
               gsem):
    c = lax.axis_index("c")
    s = lax.axis_index("s")

    zeros16 = jnp.zeros((16,), jnp.float32)

    def _zb(i, _):
        rows_v[i // 8, pl.ds((i % 8) * 16, 16)] = zeros16
        return 0

    lax.fori_loop(0, C * D // 16, _zb, 0)

    def _zc(k, _):
        pltpu.sync_copy(rows_v,
                        acc_sp.at[pl.ds(s * ROWS_PER_TILE + k * C, C)])
        return 0

    lax.fori_loop(0, ROWS_PER_TILE // C, _zc, 0)

    plsc.subcore_barrier()

    def _body(k, _):
        i = s + k * NS
        off = (c * HALF + i) * C
        pltpu.sync_copy(src_hbm.at[pl.ds(off, C)], sidx)
        pltpu.sync_copy(dst_hbm.at[pl.ds(off, C)], didx)
        pltpu.async_copy(hs_hbm.at[sidx], rows_v, gsem).wait()
        pltpu.sync_copy(rows_v, acc_sp.at[didx], add=True)
        return 0

    nmine = (HALF - s + NS - 1) // NS
    lax.fori_loop(0, nmine, _body, 0)

    plsc.subcore_barrier()

    def _wb(k, _):
        r0 = s * ROWS_PER_TILE + k * C
        pltpu.sync_copy(acc_sp.at[pl.ds(r0, C)], rows_v)
        pltpu.sync_copy(rows_v, out_hbm.at[pl.ds(c * NPAD + r0, C)])
        return 0

    lax.fori_loop(0, ROWS_PER_TILE // C, _wb, 0)


@functools.partial(
    pl.kernel,
    out_type=jax.ShapeDtypeStruct((NC * NPAD, D), jnp.float32),
    mesh=_MESH,
    scratch_types=[
        pltpu.VMEM_SHARED((NPAD, D), jnp.float32),
        pltpu.VMEM((C, D), jnp.float32),
        pltpu.VMEM((C,), jnp.int32),
        pltpu.VMEM((C,), jnp.int32),
        pltpu.SemaphoreType.DMA,
    ],
    name="sc_edge_scatter",
)
def _sc_edge(hs_hbm, src_hbm, dst_hbm, out_hbm, acc_sp, rows_v, sidx, didx,
             gsem):
    _edge_body(hs_hbm, src_hbm, dst_hbm, out_hbm, acc_sp, rows_v, sidx, didx,
               gsem)


def _tc_mm_body(x_ref, w_ref, o_ref):
    o_ref[...] = jnp.dot(x_ref[...], w_ref[...],
                         preferred_element_type=jnp.float32)


def _tc_mm(x, w):
    return pl.pallas_call(
        _tc_mm_body,
        out_shape=jax.ShapeDtypeStruct((x.shape[0], w.shape[1]), jnp.float32),
    )(x, w)


def _dinv_of(cnt_ref):
    deg = 1.0 + cnt_ref[0, :] + cnt_ref[1, :]
    return lax.rsqrt(deg)[:, None]


def _tc_scale_body(h_ref, cnt_ref, hs_ref):
    hs_ref[...] = h_ref[...] * _dinv_of(cnt_ref)


def _tc_scale(h, cnt2):
    return pl.pallas_call(
        _tc_scale_body,
        out_shape=jax.ShapeDtypeStruct((N, D), jnp.float32),
    )(h, cnt2)


def _tc_mid_body(acc_ref, hs_ref, cnt_ref, b_ref, w_ref, o_ref):
    dinv = _dinv_of(cnt_ref)
    g = dinv * (acc_ref[0:N, :] + acc_ref[NPAD:NPAD + N, :] + hs_ref[...]) \
        + b_ref[...]
    z = jnp.maximum(g, 0.0)
    h2 = jnp.dot(z, w_ref[...], preferred_element_type=jnp.float32)
    o_ref[...] = h2 * dinv


def _tc_mid(acc, hs1, cnt2, b1, w2):
    return pl.pallas_call(
        _tc_mid_body,
        out_shape=jax.ShapeDtypeStruct((N, D), jnp.float32),
    )(acc, hs1, cnt2, b1, w2)


def _tc_fin_body(acc_ref, hs_ref, cnt_ref, b_ref, wv_ref, bv_ref, wt_ref,
                 bt_ref, h_ref, xv_ref, xt_ref):
    dinv = _dinv_of(cnt_ref)
    h = dinv * (acc_ref[0:N, :] + acc_ref[NPAD:NPAD + N, :] + hs_ref[...]) \
        + b_ref[...]
    h_ref[...] = h
    xv_ref[...] = jnp.maximum(
        jnp.dot(h, wv_ref[...], preferred_element_type=jnp.float32)
        + bv_ref[...], 0.0)
    xt_ref[...] = jnp.maximum(
        jnp.dot(h, wt_ref[...], preferred_element_type=jnp.float32)
        + bt_ref[...], 0.0)


def _tc_fin(acc, hs2, cnt2, b2, wv, bv, wt, bt):
    return pl.pallas_call(
        _tc_fin_body,
        out_shape=(
            jax.ShapeDtypeStruct((N, D), jnp.float32),
            jax.ShapeDtypeStruct((N, D), jnp.float32),
            jax.ShapeDtypeStruct((N, D), jnp.float32),
        ),
    )(acc, hs2, cnt2, b2, wv, bv, wt, bt)


def kernel(x, edge_index, W1, b1, W2, b2, Wv, bv, Wt, bt):
    src = edge_index[0]
    dst = edge_index[1]
    b1r = b1.reshape(1, D)
    b2r = b2.reshape(1, D)
    bvr = bv.reshape(1, D)
    btr = bt.reshape(1, D)

    cnt2 = _sc_cnt(dst).reshape(NC, N)
    h1 = _tc_mm(x, W1)
    hs1 = _tc_scale(h1, cnt2)
    acc1 = _sc_edge(hs1, src, dst)
    hs2 = _tc_mid(acc1, hs1, cnt2, b1r, W2)
    acc2 = _sc_edge(hs2, src, dst)
    h, xv, xt = _tc_fin(acc2, hs2, cnt2, b2r, Wv, bvr, Wt, btr)
    return (h, xv, xt)
